# Initial kernel scaffold; baseline (speedup 1.0000x reference)
import jax
import jax.numpy as jnp
from jax.experimental import pallas as pl


def kernel(labels, mask):
    raise NotImplementedError("write your pallas kernel here")


# speedup vs baseline: 206.6436x; 206.6436x over previous
import functools

import jax
import jax.numpy as jnp
from jax.experimental import pallas as pl
from jax.experimental.pallas import tpu as pltpu

_NUM_CLASSES = 2
_CLIPMIN = 0.05
_CLIPMAX = 0.95

_ROWS = 8192
_COLS = 4096
_BLK = 512


def _pass1_body(labels_ref, mask_ref, acc_ref):
    i = pl.program_id(0)

    @pl.when(i == 0)
    def _init():
        acc_ref[...] = jnp.zeros_like(acc_ref)

    m = mask_ref[...]
    lab = labels_ref[...]
    sel = (m > 0.0).astype(jnp.float32)
    s_mask = jnp.sum(m)
    c1 = jnp.sum(sel * lab.astype(jnp.float32))
    csel = jnp.sum(sel)
    lane = jax.lax.broadcasted_iota(jnp.int32, (1, 128), 1)
    pv = (
        jnp.where(lane == 0, s_mask, 0.0)
        + jnp.where(lane == 1, c1, 0.0)
        + jnp.where(lane == 2, csel, 0.0)
    )
    acc_ref[...] += pv


def _pass2_body(acc_ref, labels_ref, mask_ref, out_ref):
    masked_in = acc_ref[0, 0]
    c1 = acc_ref[0, 1]
    csel = acc_ref[0, 2]
    c0 = csel - c1

    inv_n = 1.0 / float(_NUM_CLASSES)

    def weight(c):
        frac = jnp.where(masked_in > 0.0, c / masked_in, 0.0)
        frac = jnp.clip(frac, _CLIPMIN, _CLIPMAX)
        w = inv_n / frac
        return jnp.where(c > 0.0, w, 0.0)

    w0 = weight(c0)
    w1 = weight(c1)
    m = mask_ref[...]
    lab = labels_ref[...]
    out_ref[...] = m * jnp.where(lab == 1, w1, w0)


@jax.jit
def kernel(labels, mask):
    grid = _ROWS // _BLK
    acc = pl.pallas_call(
        _pass1_body,
        grid=(grid,),
        in_specs=[
            pl.BlockSpec((_BLK, _COLS), lambda i: (i, 0)),
            pl.BlockSpec((_BLK, _COLS), lambda i: (i, 0)),
        ],
        out_specs=pl.BlockSpec((1, 128), lambda i: (0, 0)),
        out_shape=jax.ShapeDtypeStruct((1, 128), jnp.float32),
    )(labels, mask)

    out = pl.pallas_call(
        _pass2_body,
        grid=(grid,),
        in_specs=[
            pl.BlockSpec((1, 128), lambda i: (0, 0)),
            pl.BlockSpec((_BLK, _COLS), lambda i: (i, 0)),
            pl.BlockSpec((_BLK, _COLS), lambda i: (i, 0)),
        ],
        out_specs=pl.BlockSpec((_BLK, _COLS), lambda i: (i, 0)),
        out_shape=jax.ShapeDtypeStruct((_ROWS, _COLS), jnp.float32),
    )(acc, labels, mask)
    return out
